# baseline (speedup 1.0000x reference)
import jax
import jax.numpy as jnp
from jax import lax
from jax.experimental import pallas as pl
from jax.experimental.pallas import tpu as pltpu
from jax.experimental.pallas import tpu_sc as plsc

N_NODES = 10000
N_EDGES = 320000
D = 128
DH = D // 2

NC = 2
NS = 16

CHUNK = 128
NCHUNK = 158
E_PAD = NS * NCHUNK * CHUNK
NPAD = 10112
ROWS_PER_TILE = NPAD // NS


def _scatter_gather_kernel(x2_hbm, src_hbm, dst_hbm, zero_hbm, h2_hbm,
                           src_v, dst_v, buf0, buf1, acc, sem0, sem1):
    c = lax.axis_index("c")
    s = lax.axis_index("s")

    pltpu.sync_copy(src_hbm.at[c, s], src_v)
    pltpu.sync_copy(dst_hbm.at[s], dst_v)

    r0 = s * ROWS_PER_TILE
    pltpu.sync_copy(zero_hbm.at[pl.ds(r0, ROWS_PER_TILE)],
                    acc.at[pl.ds(r0, ROWS_PER_TILE)])
    plsc.subcore_barrier()

    bufs = (buf0, buf1)
    sems = (sem0, sem1)

    pltpu.async_copy(x2_hbm.at[src_v.at[0]], buf0, sem0)
    pltpu.async_copy(x2_hbm.at[src_v.at[1]], buf1, sem1)

    def body(i2, carry):
        for b in range(2):
            j = i2 * 2 + b
            pltpu.make_async_copy(x2_hbm.at[src_v.at[j]], bufs[b],
                                  sems[b]).wait()
            pltpu.sync_copy(bufs[b], acc.at[dst_v.at[j]], add=True)
            jn = lax.rem(j + 2, NCHUNK)
            pltpu.async_copy(x2_hbm.at[src_v.at[jn]], bufs[b], sems[b])
        return carry

    lax.fori_loop(0, NCHUNK // 2, body, 0)

    for b in range(2):
        pltpu.make_async_copy(x2_hbm.at[src_v.at[0]], bufs[b], sems[b]).wait()

    plsc.subcore_barrier()
    pltpu.sync_copy(acc.at[pl.ds(r0, ROWS_PER_TILE)],
                    h2_hbm.at[c, pl.ds(r0, ROWS_PER_TILE)])


@jax.jit
def _segment_sum_sc(x2, src2, dst, zero):
    mesh = plsc.VectorSubcoreMesh(core_axis_name="c", subcore_axis_name="s")
    return pl.kernel(
        _scatter_gather_kernel,
        out_type=jax.ShapeDtypeStruct((NC, NPAD, DH), jnp.float32),
        mesh=mesh,
        compiler_params=pltpu.CompilerParams(use_tc_tiling_on_sc=False),
        scratch_types=[
            pltpu.VMEM((NCHUNK, CHUNK), jnp.int32),
            pltpu.VMEM((NCHUNK, CHUNK), jnp.int32),
            pltpu.VMEM((CHUNK, DH), jnp.float32),
            pltpu.VMEM((CHUNK, DH), jnp.float32),
            pltpu.VMEM_SHARED((NPAD, DH), jnp.float32),
            pltpu.SemaphoreType.DMA,
            pltpu.SemaphoreType.DMA,
        ],
    )(x2, src2, dst, zero)


def _linear_body(h2_ref, w2_ref, b_ref, o_ref):
    o_ref[...] = (
        lax.dot_general(h2_ref[0], w2_ref[0], (((1,), (1,)), ((), ())),
                        preferred_element_type=jnp.float32)
        + lax.dot_general(h2_ref[1], w2_ref[1], (((1,), (1,)), ((), ())),
                          preferred_element_type=jnp.float32)
        + b_ref[...])


@jax.jit
def _linear_tc(h2, W2, b2):
    blk = 1000
    grid = N_NODES // blk
    return pl.pallas_call(
        _linear_body,
        grid=(grid,),
        in_specs=[
            pl.BlockSpec((NC, blk, DH), lambda i: (0, i, 0)),
            pl.BlockSpec((NC, D, DH), lambda i: (0, 0, 0)),
            pl.BlockSpec((1, D), lambda i: (0, 0)),
        ],
        out_specs=pl.BlockSpec((blk, D), lambda i: (i, 0)),
        out_shape=jax.ShapeDtypeStruct((N_NODES, D), jnp.float32),
    )(h2, W2, b2)


def kernel(inputs, edge_index, W, b):
    n_pad = E_PAD - N_EDGES
    rpad = NPAD - N_NODES
    x2 = jnp.concatenate([
        jnp.pad(inputs[:, :DH], ((0, rpad), (0, 0))),
        jnp.pad(inputs[:, DH:], ((0, rpad), (0, 0))),
    ])
    src = jnp.concatenate(
        [edge_index[0], jnp.zeros((n_pad,), jnp.int32)]
    ).reshape(NS, NCHUNK, CHUNK)
    src2 = jnp.stack([src, src + NPAD])
    dst = jnp.concatenate(
        [edge_index[1], jnp.full((n_pad,), NPAD - 1, jnp.int32)]
    ).reshape(NS, NCHUNK, CHUNK)
    zero = jnp.zeros((NPAD, DH), jnp.float32)
    h2 = _segment_sum_sc(x2, src2, dst, zero)
    W2 = jnp.stack([W[:, :DH], W[:, DH:]])
    return _linear_tc(h2, W2, b.reshape(1, D))
